# Initial kernel scaffold; baseline (speedup 1.0000x reference)
import jax
import jax.numpy as jnp
from jax.experimental import pallas as pl


def kernel(joint):
    raise NotImplementedError("write your pallas kernel here")


# speedup vs baseline: 129.2776x; 129.2776x over previous
import functools

import jax
import jax.numpy as jnp
from jax.experimental import pallas as pl

NUM_JOINTS = 21
IMG_SIZE = 256
G_SIZE = 11
G_SIGMA = 3.0
BATCH = 32

_C = BATCH * NUM_JOINTS
_CPB = 16
_INV2S2 = 1.0 / (2.0 * G_SIGMA * G_SIGMA)
_R = (G_SIZE - 1) // 2
_R2 = float(_R * _R)


def _stamp_kernel(xy_ref, out_ref):
    S = IMG_SIZE
    xy = xy_ref[:, 0:2]
    uv = (xy * 0.25 + 0.5) * (S - 1)
    iuv = jnp.clip(jnp.round(uv), 0.0, float(S - 1))
    x = iuv[:, 0:1]
    y = iuv[:, 1:2]
    pos = jax.lax.broadcasted_iota(jnp.int32, (_CPB, S), 1).astype(jnp.float32)
    dr = pos - y
    dc = pos - x
    rv = jnp.where(dr * dr <= _R2, jnp.exp(-(dr * dr) * _INV2S2), 0.0)
    cv = jnp.where(dc * dc <= _R2, jnp.exp(-(dc * dc) * _INV2S2), 0.0)
    out_ref[...] = rv[:, :, None] * cv[:, None, :]


@functools.partial(jax.jit, static_argnames=("interpret",))
def _heatmap(xy, interpret=False):
    S = IMG_SIZE
    grid = (_C // _CPB,)
    return pl.pallas_call(
        _stamp_kernel,
        grid=grid,
        in_specs=[pl.BlockSpec((_CPB, 4), lambda i: (i, 0))],
        out_specs=pl.BlockSpec((_CPB, S, S), lambda i: (i, 0, 0)),
        out_shape=jax.ShapeDtypeStruct((_C, S, S), jnp.float32),
        interpret=interpret,
    )(xy)


def kernel(joint, interpret=False):
    B, J, _ = joint.shape
    xy = jnp.pad(joint.reshape(B * J, 3), ((0, 0), (0, 1)))
    out = _heatmap(xy, interpret=interpret)
    return out.reshape(B, J, IMG_SIZE, IMG_SIZE)
